# baseline (speedup 1.0000x reference)
import jax
import jax.numpy as jnp
from jax.experimental import pallas as pl
from jax.experimental.pallas import tpu as pltpu

_BM = 80


def _sanbet_kernel(w_ref, b_ref, adj_ref, inp_ref, out_ref):
    acc = jax.lax.dot_general(
        adj_ref[...], inp_ref[...], (((1,), (0,)), ((), ())),
        preferred_element_type=jnp.float32,
        precision=jax.lax.Precision.DEFAULT,
    )
    out_ref[...] = acc * w_ref[0, 0] + b_ref[0, 0]


def kernel(inp, adj, weight, bias):
    n, d = inp.shape
    w2 = weight.reshape(1, 1)
    b2 = bias.reshape(1, 1)
    grid = (n // _BM,)
    return pl.pallas_call(
        _sanbet_kernel,
        grid=grid,
        in_specs=[
            pl.BlockSpec((1, 1), lambda i: (0, 0)),
            pl.BlockSpec((1, 1), lambda i: (0, 0)),
            pl.BlockSpec((_BM, n), lambda i: (i, 0)),
            pl.BlockSpec((n, d), lambda i: (0, 0)),
        ],
        out_specs=pl.BlockSpec((_BM, d), lambda i: (i, 0)),
        out_shape=jax.ShapeDtypeStruct((n, d), jnp.float32),
        compiler_params=pltpu.CompilerParams(
            dimension_semantics=("arbitrary",),
        ),
    )(w2, b2, adj, inp)


# speedup vs baseline: 1.2605x; 1.2605x over previous
import jax
import jax.numpy as jnp
from jax.experimental import pallas as pl
from jax.experimental.pallas import tpu as pltpu

_MID = 400


def _chunk_schedule(n):
    if n < 2400 or (n - 800) % _MID != 0:
        return [n]
    return [80, 160, 320] + [_MID] * ((n - 800) // _MID) + [160, 80]


def _make_body(sizes):
    offs = []
    o = 0
    for s in sizes:
        offs.append(o)
        o += s

    def body(w_ref, b_ref, inp_ref, adj_hbm, out_ref, buf0, buf1, sem0, sem1):
        bufs = (buf0, buf1)
        sems = (sem0, sem1)

        def copy(i):
            return pltpu.make_async_copy(
                adj_hbm.at[pl.ds(offs[i], sizes[i]), :],
                bufs[i % 2].at[pl.ds(0, sizes[i]), :],
                sems[i % 2],
            )

        pending = copy(0)
        pending.start()
        for i in range(len(sizes)):
            if i + 1 < len(sizes):
                nxt = copy(i + 1)
                nxt.start()
            pending.wait()
            a = bufs[i % 2][pl.ds(0, sizes[i]), :]
            acc = jax.lax.dot_general(
                a, inp_ref[...], (((1,), (0,)), ((), ())),
                preferred_element_type=jnp.float32,
                precision=jax.lax.Precision.DEFAULT,
            )
            out_ref[pl.ds(offs[i], sizes[i]), :] = acc * w_ref[0, 0] + b_ref[0, 0]
            if i + 1 < len(sizes):
                pending = nxt

    return body


def kernel(inp, adj, weight, bias):
    n, d = inp.shape
    w2 = weight.reshape(1, 1)
    b2 = bias.reshape(1, 1)
    sizes = _chunk_schedule(n)
    bufrows = max(sizes)
    return pl.pallas_call(
        _make_body(sizes),
        in_specs=[
            pl.BlockSpec(memory_space=pltpu.MemorySpace.VMEM),
            pl.BlockSpec(memory_space=pltpu.MemorySpace.VMEM),
            pl.BlockSpec(memory_space=pltpu.MemorySpace.VMEM),
            pl.BlockSpec(memory_space=pltpu.MemorySpace.HBM),
        ],
        out_specs=pl.BlockSpec(memory_space=pltpu.MemorySpace.VMEM),
        out_shape=jax.ShapeDtypeStruct((n, d), jnp.float32),
        scratch_shapes=[
            pltpu.VMEM((bufrows, n), jnp.float32),
            pltpu.VMEM((bufrows, n), jnp.float32),
            pltpu.SemaphoreType.DMA,
            pltpu.SemaphoreType.DMA,
        ],
    )(w2, b2, inp, adj)


# speedup vs baseline: 1.3606x; 1.0794x over previous
import jax
import jax.numpy as jnp
from jax.experimental import pallas as pl
from jax.experimental.pallas import tpu as pltpu

_BM = 200


def _sanbet_kernel(w_ref, b_ref, adj_ref, inp_ref, out_ref):
    acc = jax.lax.dot_general(
        adj_ref[...], inp_ref[...], (((1,), (0,)), ((), ())),
        preferred_element_type=jnp.float32,
        precision=jax.lax.Precision.DEFAULT,
    )
    out_ref[...] = acc * w_ref[0, 0] + b_ref[0, 0]


def kernel(inp, adj, weight, bias):
    n, d = inp.shape
    w2 = weight.reshape(1, 1)
    b2 = bias.reshape(1, 1)
    grid = (n // _BM,)
    return pl.pallas_call(
        _sanbet_kernel,
        grid=grid,
        in_specs=[
            pl.BlockSpec((1, 1), lambda i: (0, 0)),
            pl.BlockSpec((1, 1), lambda i: (0, 0)),
            pl.BlockSpec((_BM, n), lambda i: (i, 0)),
            pl.BlockSpec((n, d), lambda i: (0, 0)),
        ],
        out_specs=pl.BlockSpec((_BM, d), lambda i: (i, 0)),
        out_shape=jax.ShapeDtypeStruct((n, d), jnp.float32),
        compiler_params=pltpu.CompilerParams(
            dimension_semantics=("arbitrary",),
        ),
    )(w2, b2, adj, inp)
